# baseline (speedup 1.0000x reference)
import functools

import jax
import jax.numpy as jnp
from jax import lax
from jax.experimental import pallas as pl
from jax.experimental.pallas import tpu as pltpu
from jax.experimental.pallas import tpu_sc as plsc

NUM_CORES = 2
NUM_SUBCORES = 16
NUM_WORKERS = NUM_CORES * NUM_SUBCORES
LANES = 16
OBUF = 4
IBUF = 2
SLAB = 8


@functools.lru_cache(maxsize=None)
def _make_kernel(batch: int, hist: int, vocab: int, dim: int):
    rounds = dim // NUM_WORKERS
    bt = batch // 128
    blocks = batch // LANES
    mesh = plsc.VectorSubcoreMesh(
        core_axis_name="c", subcore_axis_name="s")

    @functools.partial(
        pl.kernel,
        mesh=mesh,
        out_type=jax.ShapeDtypeStruct((hist, dim // 8, bt, 8, 128),
                                      jnp.float32),
        scratch_types=[
            pltpu.VMEM((vocab,), jnp.float32),
            pltpu.VMEM((IBUF, batch), jnp.int32),
            pltpu.VMEM((OBUF, bt, 128), jnp.float32),
            pltpu.VMEM_SHARED((2, SLAB, batch), jnp.int32),
            [pltpu.SemaphoreType.DMA] * IBUF,
            [pltpu.SemaphoreType.DMA] * OBUF,
            pltpu.SemaphoreType.DMA,
        ],
        compiler_params=pltpu.CompilerParams(use_tc_tiling_on_sc=True,
                                             needs_layout_passes=False),
    )
    def emb_kernel(idx_t, table_t, out_k, feat_v, idx_v, out_v, idx_sh,
                   isems, osems, fsem):
        wid = lax.axis_index("s") * NUM_CORES + lax.axis_index("c")
        is_filler = lax.axis_index("s") == 0
        n_slabs = hist // SLAB

        for r in range(rounds):
            d = wid * rounds + r
            dt = d // 8
            ds = d % 8
            pltpu.sync_copy(table_t.at[d, :], feat_v)
            @pl.when(is_filler)
            def _():
                pltpu.sync_copy(idx_t.at[pl.ds(0, SLAB), :], idx_sh.at[0])
            plsc.subcore_barrier()
            for b in range(IBUF):
                pltpu.async_copy(idx_sh.at[0, b, :], idx_v.at[b, :],
                                 isems[b])

            @pl.loop(0, n_slabs)
            def _(S):
                sp = S % 2
                sn = (S + 1) % 2
                more = S + 1 < n_slabs

                @pl.when(jnp.logical_and(is_filler, more))
                def _():
                    pltpu.async_copy(
                        idx_t.at[pl.ds((S + 1) * SLAB, SLAB), :],
                        idx_sh.at[sn], fsem)

                for hb in range(SLAB):
                    b = hb % OBUF
                    ip = hb % IBUF
                    h = S * SLAB + hb
                    pltpu.make_async_copy(
                        idx_sh.at[sp, hb, :], idx_v.at[ip, :],
                        isems[ip]).wait()

                    def _drain():
                        pltpu.make_async_copy(
                            out_v.at[b], out_k.at[0, 0, :, 0, :],
                            osems[b]).wait()

                    if r > 0:
                        _drain()
                    else:
                        pl.when(h >= OBUF)(_drain)

                    @plsc.parallel_loop(0, blocks, unroll=16)
                    def _(j):
                        idx = idx_v[ip, pl.ds(j * LANES, LANES)]
                        out_v[b, j // 8, pl.ds((j % 8) * LANES, LANES)] = (
                            plsc.load_gather(feat_v, [idx]))

                    pltpu.async_copy(
                        out_v.at[b], out_k.at[h, dt, :, ds, :], osems[b])

                    if hb < SLAB - IBUF:
                        pltpu.async_copy(
                            idx_sh.at[sp, hb + IBUF, :], idx_v.at[ip, :],
                            isems[ip])

                @pl.when(jnp.logical_and(is_filler, more))
                def _():
                    pltpu.make_async_copy(
                        idx_t.at[pl.ds(0, SLAB), :], idx_sh.at[sn],
                        fsem).wait()
                plsc.subcore_barrier()
                for b in range(IBUF):
                    @pl.when(more)
                    def _():
                        pltpu.async_copy(idx_sh.at[sn, b, :],
                                         idx_v.at[b, :], isems[b])

            if r == rounds - 1:
                for b in range(OBUF):
                    pltpu.make_async_copy(
                        out_v.at[b], out_k.at[0, 0, :, 0, :],
                        osems[b]).wait()

    return emb_kernel


def kernel(index, table):
    batch, hist = index.shape
    vocab, dim = table.shape
    out_k = _make_kernel(batch, hist, vocab, dim)(index.T, table.T)
    return out_k.transpose(2, 4, 0, 1, 3).reshape(batch, hist, dim)


# speedup vs baseline: 1.0575x; 1.0575x over previous
import functools

import jax
import jax.numpy as jnp
from jax import lax
from jax.experimental import pallas as pl
from jax.experimental.pallas import tpu as pltpu
from jax.experimental.pallas import tpu_sc as plsc

NUM_CORES = 2
NUM_SUBCORES = 16
NUM_WORKERS = NUM_CORES * NUM_SUBCORES
LANES = 16
OBUF = 4
IBUF = 2
SLAB = 8


@functools.lru_cache(maxsize=None)
def _make_kernel(batch: int, hist: int, vocab: int, dim: int):
    rounds = dim // NUM_WORKERS
    bt = batch // 128
    blocks = batch // LANES
    mesh = plsc.VectorSubcoreMesh(
        core_axis_name="c", subcore_axis_name="s")

    @functools.partial(
        pl.kernel,
        mesh=mesh,
        out_type=jax.ShapeDtypeStruct((hist, dim // 8, bt, 8, 128),
                                      jnp.float32),
        scratch_types=[
            pltpu.VMEM((vocab,), jnp.float32),
            pltpu.VMEM((IBUF, batch), jnp.int32),
            pltpu.VMEM((OBUF, bt, 128), jnp.float32),
            pltpu.VMEM_SHARED((2, SLAB, batch), jnp.int32),
            [pltpu.SemaphoreType.DMA] * IBUF,
            [pltpu.SemaphoreType.DMA] * OBUF,
            pltpu.SemaphoreType.DMA,
        ],
        compiler_params=pltpu.CompilerParams(use_tc_tiling_on_sc=False,
                                             needs_layout_passes=False),
    )
    def emb_kernel(idx_t, table_t, out_k, feat_v, idx_v, out_v, idx_sh,
                   isems, osems, fsem):
        wid = lax.axis_index("s") * NUM_CORES + lax.axis_index("c")
        is_filler = lax.axis_index("s") == 0
        n_slabs = hist // SLAB

        for r in range(rounds):
            d = wid * rounds + r
            dt = d // 8
            ds = d % 8
            pltpu.sync_copy(table_t.at[d, :], feat_v)
            @pl.when(is_filler)
            def _():
                pltpu.sync_copy(idx_t.at[pl.ds(0, SLAB), :], idx_sh.at[0])
            plsc.subcore_barrier()
            for b in range(IBUF):
                pltpu.async_copy(idx_sh.at[0, b, :], idx_v.at[b, :],
                                 isems[b])

            @pl.loop(0, n_slabs)
            def _(S):
                sp = S % 2
                sn = (S + 1) % 2
                more = S + 1 < n_slabs

                @pl.when(jnp.logical_and(is_filler, more))
                def _():
                    pltpu.async_copy(
                        idx_t.at[pl.ds((S + 1) * SLAB, SLAB), :],
                        idx_sh.at[sn], fsem)

                for hb in range(SLAB):
                    b = hb % OBUF
                    ip = hb % IBUF
                    h = S * SLAB + hb
                    pltpu.make_async_copy(
                        idx_sh.at[sp, hb, :], idx_v.at[ip, :],
                        isems[ip]).wait()

                    def _drain():
                        pltpu.make_async_copy(
                            out_v.at[b], out_k.at[0, 0, :, 0, :],
                            osems[b]).wait()

                    if r > 0:
                        _drain()
                    else:
                        pl.when(h >= OBUF)(_drain)

                    @plsc.parallel_loop(0, blocks, unroll=32)
                    def _(j):
                        idx = idx_v[ip, pl.ds(j * LANES, LANES)]
                        out_v[b, j // 8, pl.ds((j % 8) * LANES, LANES)] = (
                            plsc.load_gather(feat_v, [idx]))

                    pltpu.async_copy(
                        out_v.at[b], out_k.at[h, dt, :, ds, :], osems[b])

                    if hb < SLAB - IBUF:
                        pltpu.async_copy(
                            idx_sh.at[sp, hb + IBUF, :], idx_v.at[ip, :],
                            isems[ip])

                @pl.when(jnp.logical_and(is_filler, more))
                def _():
                    pltpu.make_async_copy(
                        idx_t.at[pl.ds(0, SLAB), :], idx_sh.at[sn],
                        fsem).wait()
                plsc.subcore_barrier()
                for b in range(IBUF):
                    @pl.when(more)
                    def _():
                        pltpu.async_copy(idx_sh.at[sn, b, :],
                                         idx_v.at[b, :], isems[b])

            if r == rounds - 1:
                for b in range(OBUF):
                    pltpu.make_async_copy(
                        out_v.at[b], out_k.at[0, 0, :, 0, :],
                        osems[b]).wait()

    return emb_kernel


def kernel(index, table):
    batch, hist = index.shape
    vocab, dim = table.shape
    out_k = _make_kernel(batch, hist, vocab, dim)(index.T, table.T)
    return out_k.transpose(2, 4, 0, 1, 3).reshape(batch, hist, dim)
